# baseline (speedup 1.0000x reference)
import functools

import jax
import jax.numpy as jnp
from jax import lax
from jax.experimental import pallas as pl
from jax.experimental.pallas import tpu as pltpu
from jax.experimental.pallas import tpu_sc as plsc

_D = 32
_LB = 128
_NBUF = 16


def _build(B, V, num_cores, num_subcores):
    NW = num_cores * num_subcores
    b_per_w = B // NW
    n_groups = b_per_w // _NBUF
    mesh = plsc.VectorSubcoreMesh(core_axis_name="c", subcore_axis_name="s")

    @functools.partial(
        pl.kernel,
        mesh=mesh,
        compiler_params=pltpu.CompilerParams(needs_layout_passes=False),
        out_type=jax.ShapeDtypeStruct((2 * _D, B), jnp.float32),
        scratch_types=[
            pltpu.VMEM((b_per_w + 16,), jnp.int32),
            pltpu.VMEM((b_per_w + 16,), jnp.int32),
            pltpu.VMEM((_NBUF, _D, _LB), jnp.float32),
            pltpu.VMEM((2 * _D, b_per_w), jnp.float32),
        ] + [pltpu.SemaphoreType.DMA] * _NBUF,
    )
    def two_tower_gather(uidx_hbm, iidx_hbm, utab_hbm, itab_hbm, out_hbm,
                         uidx_sm, iidx_sm, ring_v, gbuf_v, *sems):
        wid = lax.axis_index("s") * num_cores + lax.axis_index("c")
        base = wid * b_per_w
        lane = lax.broadcasted_iota(jnp.int32, (16,), 0)
        pltpu.sync_copy(uidx_hbm.at[pl.ds(base, b_per_w)],
                        uidx_sm.at[pl.ds(0, b_per_w)])
        pltpu.sync_copy(iidx_hbm.at[pl.ds(base, b_per_w)],
                        iidx_sm.at[pl.ds(0, b_per_w)])

        def run_tower(idx_sm, tab_hbm, row0):
            def sidx(j):
                return idx_sm[pl.ds(j, 16)][0]

            def fire(j, slot, sem):
                off = pl.multiple_of(
                    lax.shift_left(
                        lax.shift_right_logical(sidx(j), 7), 7), _LB)
                pltpu.async_copy(
                    tab_hbm.at[pl.ds(0, _D), pl.ds(off, _LB)],
                    ring_v.at[slot], sem)

            def extract(j, slot):
                c = lax.bitwise_and(sidx(j), _LB - 1)
                cs = lax.broadcast(c, (16,))
                js = lax.broadcast(j, (16,))
                for h in range(_D // 16):
                    rows = lane + h * 16
                    v = plsc.load_gather(ring_v.at[slot], [rows, cs])
                    plsc.store_scatter(gbuf_v, [rows + row0, js], v)

            for slot in range(_NBUF):
                fire(slot, slot, sems[slot])

            def group(g, carry):
                for slot in range(_NBUF):
                    j = g * _NBUF + slot
                    pltpu.make_async_copy(
                        tab_hbm.at[pl.ds(0, _D), pl.ds(0, _LB)],
                        ring_v.at[slot], sems[slot]).wait()
                    extract(j, slot)
                    nj = j + _NBUF

                    @pl.when(nj < b_per_w)
                    def _fire():
                        fire(nj, slot, sems[slot])
                return carry
            lax.fori_loop(0, n_groups, group, 0)

        run_tower(uidx_sm, utab_hbm, 0)
        run_tower(iidx_sm, itab_hbm, _D)

        pltpu.sync_copy(gbuf_v, out_hbm.at[:, pl.ds(base, b_per_w)])

    return two_tower_gather


def kernel(user_idx, item_idx, user_table, item_table):
    B = user_idx.shape[0]
    V, D = user_table.shape
    assert D == _D
    ut = user_table.T
    it = item_table.T
    info = plsc.get_sparse_core_info()
    fn = _build(B, V, info.num_cores, info.num_subcores)
    out = fn(user_idx, item_idx, ut, it)
    return out.reshape(2, _D, B).transpose(0, 2, 1)


# speedup vs baseline: 1.0272x; 1.0272x over previous
import functools

import jax
import jax.numpy as jnp
from jax import lax
from jax.experimental import pallas as pl
from jax.experimental.pallas import tpu as pltpu
from jax.experimental.pallas import tpu_sc as plsc

_D = 32
_LB = 128
_NBUF = 8


def _build(B, V, num_cores, num_subcores):
    NW = num_cores * num_subcores
    b_per_w = B // NW
    n_groups = b_per_w // _NBUF
    mesh = plsc.VectorSubcoreMesh(core_axis_name="c", subcore_axis_name="s")

    @functools.partial(
        pl.kernel,
        mesh=mesh,
        compiler_params=pltpu.CompilerParams(needs_layout_passes=False),
        out_type=jax.ShapeDtypeStruct((2 * _D, B), jnp.float32),
        scratch_types=[
            pltpu.VMEM((b_per_w + 16,), jnp.int32),
            pltpu.VMEM((b_per_w + 16,), jnp.int32),
            pltpu.VMEM((_NBUF, _D, _LB), jnp.float32),
            pltpu.VMEM((2 * _D, b_per_w), jnp.float32),
        ] + [pltpu.SemaphoreType.DMA] * _NBUF,
    )
    def two_tower_gather(uidx_hbm, iidx_hbm, utab_hbm, itab_hbm, out_hbm,
                         uidx_sm, iidx_sm, ring_v, gbuf_v, *sems):
        wid = lax.axis_index("s") * num_cores + lax.axis_index("c")
        base = wid * b_per_w
        lane = lax.broadcasted_iota(jnp.int32, (16,), 0)
        pltpu.sync_copy(uidx_hbm.at[pl.ds(base, b_per_w)],
                        uidx_sm.at[pl.ds(0, b_per_w)])
        pltpu.sync_copy(iidx_hbm.at[pl.ds(base, b_per_w)],
                        iidx_sm.at[pl.ds(0, b_per_w)])

        def run_tower(idx_sm, tab_hbm, row0):
            def sidx(j):
                return idx_sm[pl.ds(j, 16)][0]

            def fire(j, slot, sem):
                off = pl.multiple_of(
                    lax.shift_left(
                        lax.shift_right_logical(sidx(j), 7), 7), _LB)
                pltpu.async_copy(
                    tab_hbm.at[pl.ds(0, _D), pl.ds(off, _LB)],
                    ring_v.at[slot], sem)

            def extract(j, slot):
                c = lax.bitwise_and(sidx(j), _LB - 1)
                cs = lax.broadcast(c, (16,))
                js = lax.broadcast(j, (16,))
                for h in range(_D // 16):
                    rows = lane + h * 16
                    v = plsc.load_gather(ring_v.at[slot], [rows, cs])
                    plsc.store_scatter(gbuf_v, [rows + row0, js], v)

            for slot in range(_NBUF):
                fire(slot, slot, sems[slot])

            def group(g, carry):
                for slot in range(_NBUF):
                    j = g * _NBUF + slot
                    pltpu.make_async_copy(
                        tab_hbm.at[pl.ds(0, _D), pl.ds(0, _LB)],
                        ring_v.at[slot], sems[slot]).wait()
                    extract(j, slot)
                    nj = j + _NBUF

                    @pl.when(nj < b_per_w)
                    def _fire():
                        fire(nj, slot, sems[slot])
                return carry
            lax.fori_loop(0, n_groups, group, 0)

        run_tower(uidx_sm, utab_hbm, 0)
        run_tower(iidx_sm, itab_hbm, _D)

        pltpu.sync_copy(gbuf_v, out_hbm.at[:, pl.ds(base, b_per_w)])

    return two_tower_gather


def kernel(user_idx, item_idx, user_table, item_table):
    B = user_idx.shape[0]
    V, D = user_table.shape
    assert D == _D
    ut = user_table.T
    it = item_table.T
    info = plsc.get_sparse_core_info()
    fn = _build(B, V, info.num_cores, info.num_subcores)
    out = fn(user_idx, item_idx, ut, it)
    return out.reshape(2, _D, B).transpose(0, 2, 1)
